# baseline (speedup 1.0000x reference)
import functools

import jax
import jax.numpy as jnp
from jax.experimental import pallas as pl
from jax.experimental.pallas import tpu as pltpu

_BN_EPS = 1e-5


def _fused_kernel(v1_ref, v2_ref, wall_ref, ball_ref, wwf_ref, wwb_ref,
                  bw_ref, gam_ref, bet_ref, out_ref,
                  vst, yst, y2a, sya, sca, shf, *, b, c, n):
    f32 = jnp.float32
    bf16 = jnp.bfloat16
    ph = pl.program_id(0)
    bi = pl.program_id(1)

    @pl.when(ph == 0)
    def _phase_read():
        @pl.when(bi == 0)
        def _init():
            y2a[...] = jnp.zeros_like(y2a)
            sya[...] = jnp.zeros_like(sya)

        vb = jnp.concatenate([v1_ref[0], v2_ref[0]],
                             axis=0).astype(bf16)
        vst[pl.ds(bi, 1)] = vb[None]
        proj = jnp.dot(wall_ref[...], vb,
                       preferred_element_type=f32) + ball_ref[...]
        g = proj[:c].astype(bf16)
        phi = proj[c:2 * c].astype(bf16)
        th = proj[2 * c:].astype(bf16)
        s = jax.lax.dot_general(g, phi, (((1,), (1,)), ((), ())),
                                preferred_element_type=f32) * (1.0 / n)
        y = jnp.dot(s.astype(bf16), th, preferred_element_type=f32)
        yb = y.astype(bf16)
        yst[pl.ds(bi, 1)] = yb[None]
        sya[...] += jnp.sum(yb.astype(f32), axis=1, keepdims=True)
        y2a[...] += jax.lax.dot_general(yb, yb, (((1,), (1,)), ((), ())),
                                        preferred_element_type=f32)

        @pl.when(bi == pl.num_programs(1) - 1)
        def _bn_coeffs():
            wwf = wwf_ref[...]
            bw1 = bw_ref[...]
            total = float(b * n)
            ws = jnp.dot(wwf, sya[...], preferred_element_type=f32)
            wsum = ws + total * bw1
            t1 = jnp.dot(wwf, y2a[...], preferred_element_type=f32)
            sumsq = (jnp.sum(t1 * wwf, axis=1, keepdims=True)
                     + 2.0 * bw1 * ws + total * bw1 * bw1)
            mean = wsum / total
            var = sumsq / total - mean * mean
            sc = gam_ref[...] * jax.lax.rsqrt(var + _BN_EPS)
            sca[...] = sc
            shf[...] = bet_ref[...] - mean * sc

    @pl.when(ph == 1)
    def _phase_write():
        yb = yst[pl.ds(bi, 1)][0]
        wy = jnp.dot(wwb_ref[...], yb,
                     preferred_element_type=f32) + bw_ref[...]
        vb = vst[pl.ds(bi, 1)][0]
        out_ref[0] = wy * sca[...] + shf[...] + vb.astype(f32)


def kernel(v, w_gp, b_gp, w_t, b_t, w_w, b_w, gamma, beta):
    b, d, n = v.shape
    c = w_t.shape[0]
    bf16 = jnp.bfloat16

    w_all = jnp.concatenate([w_gp, w_t], axis=0).astype(bf16)
    b_all = jnp.concatenate([b_gp, b_t], axis=0)
    ww_b = w_w.astype(bf16)

    v1_spec = pl.BlockSpec(
        (1, d // 2, n), lambda ph, bi: (jnp.where(ph == 0, bi, b - 1), 0, 0))
    v2_spec = pl.BlockSpec(
        (1, d // 2, n), lambda ph, bi: (jnp.where(ph == 0, bi, b - 1), 1, 0))
    out_spec = pl.BlockSpec(
        (1, d, n), lambda ph, bi: (jnp.where(ph == 0, 0, bi), 0, 0))
    const = lambda shape: pl.BlockSpec(shape, lambda ph, bi: (0, 0))

    out = pl.pallas_call(
        functools.partial(_fused_kernel, b=b, c=c, n=n),
        out_shape=jax.ShapeDtypeStruct((b, d, n), jnp.float32),
        grid=(2, b),
        in_specs=[v1_spec, v2_spec, const((3 * c, d)), const((3 * c, 1)),
                  const((d, c)), const((d, c)), const((d, 1)),
                  const((d, 1)), const((d, 1))],
        out_specs=out_spec,
        scratch_shapes=[
            pltpu.VMEM((b, d, n), bf16),
            pltpu.VMEM((b, c, n), bf16),
            pltpu.VMEM((c, c), jnp.float32),
            pltpu.VMEM((c, 1), jnp.float32),
            pltpu.VMEM((d, 1), jnp.float32),
            pltpu.VMEM((d, 1), jnp.float32),
        ],
        compiler_params=pltpu.CompilerParams(
            dimension_semantics=("arbitrary", "arbitrary"),
            vmem_limit_bytes=60 * 1024 * 1024),
    )(v, v, w_all, b_all, w_w, ww_b, b_w, gamma[:, None], beta[:, None])

    return out


# speedup vs baseline: 1.0386x; 1.0386x over previous
import functools

import jax
import jax.numpy as jnp
from jax.experimental import pallas as pl
from jax.experimental.pallas import tpu as pltpu

_BN_EPS = 1e-5


def _fused_kernel(v_ref, wgp_ref, bgp_ref, wt_ref, bt_ref, wwf_ref,
                  bw_ref, gam_ref, bet_ref, out_ref,
                  vst, yst, y2a, sya, sca, shf, *, b, c, n):
    f32 = jnp.float32
    bf16 = jnp.bfloat16
    ph = pl.program_id(0)
    bi = pl.program_id(1)

    @pl.when(ph == 0)
    def _phase_read():
        @pl.when(bi == 0)
        def _init():
            y2a[...] = jnp.zeros_like(y2a)
            sya[...] = jnp.zeros_like(sya)

        vb = v_ref[0].astype(bf16)
        vst[pl.ds(bi, 1)] = vb[None]
        proj = jnp.dot(wgp_ref[...].astype(bf16), vb,
                       preferred_element_type=f32) + bgp_ref[...]
        g = proj[:c].astype(bf16)
        phi = proj[c:2 * c].astype(bf16)
        th = (jnp.dot(wt_ref[...].astype(bf16), vb,
                      preferred_element_type=f32)
              + bt_ref[...]).astype(bf16)
        s = jax.lax.dot_general(g, phi, (((1,), (1,)), ((), ())),
                                preferred_element_type=f32) * (1.0 / n)
        y = jnp.dot(s.astype(bf16), th, preferred_element_type=f32)
        yb = y.astype(bf16)
        yst[pl.ds(bi, 1)] = yb[None]
        sya[...] += jnp.sum(yb.astype(f32), axis=1, keepdims=True)
        y2a[...] += jax.lax.dot_general(yb, yb, (((1,), (1,)), ((), ())),
                                        preferred_element_type=f32)

        @pl.when(bi == pl.num_programs(1) - 1)
        def _bn_coeffs():
            wwf = wwf_ref[...]
            bw1 = bw_ref[...]
            total = float(b * n)
            ws = jnp.dot(wwf, sya[...], preferred_element_type=f32)
            wsum = ws + total * bw1
            t1 = jnp.dot(wwf, y2a[...], preferred_element_type=f32)
            sumsq = (jnp.sum(t1 * wwf, axis=1, keepdims=True)
                     + 2.0 * bw1 * ws + total * bw1 * bw1)
            mean = wsum / total
            var = sumsq / total - mean * mean
            sc = gam_ref[...] * jax.lax.rsqrt(var + _BN_EPS)
            sca[...] = sc
            shf[...] = bet_ref[...] - mean * sc

    @pl.when(ph == 1)
    def _phase_write():
        yb = yst[pl.ds(bi, 1)][0]
        wy = jnp.dot(wwf_ref[...].astype(bf16), yb,
                     preferred_element_type=f32) + bw_ref[...]
        vb = vst[pl.ds(bi, 1)][0]
        out_ref[0] = wy * sca[...] + shf[...] + vb.astype(f32)


def kernel(v, w_gp, b_gp, w_t, b_t, w_w, b_w, gamma, beta):
    b, d, n = v.shape
    c = w_t.shape[0]
    bf16 = jnp.bfloat16

    v_spec = pl.BlockSpec(
        (1, d, n), lambda ph, bi: (jnp.where(ph == 0, bi, b - 1), 0, 0))
    out_spec = pl.BlockSpec(
        (1, d, n), lambda ph, bi: (jnp.where(ph == 0, 0, bi), 0, 0))
    const = lambda shape: pl.BlockSpec(shape, lambda ph, bi: (0, 0))

    out = pl.pallas_call(
        functools.partial(_fused_kernel, b=b, c=c, n=n),
        out_shape=jax.ShapeDtypeStruct((b, d, n), jnp.float32),
        grid=(2, b),
        in_specs=[v_spec, const((2 * c, d)), const((2 * c, 1)),
                  const((c, d)), const((c, 1)),
                  const((d, c)), const((d, 1)),
                  const((d, 1)), const((d, 1))],
        out_specs=out_spec,
        scratch_shapes=[
            pltpu.VMEM((b, d, n), bf16),
            pltpu.VMEM((b, c, n), bf16),
            pltpu.VMEM((c, c), jnp.float32),
            pltpu.VMEM((c, 1), jnp.float32),
            pltpu.VMEM((d, 1), jnp.float32),
            pltpu.VMEM((d, 1), jnp.float32),
        ],
        compiler_params=pltpu.CompilerParams(
            dimension_semantics=("arbitrary", "arbitrary"),
            vmem_limit_bytes=60 * 1024 * 1024),
    )(v, w_gp, b_gp, w_t, b_t, w_w, b_w, gamma[:, None], beta[:, None])

    return out
